# baseline (speedup 1.0000x reference)
import functools

import jax
import jax.numpy as jnp
from jax import lax
from jax.experimental import pallas as pl
from jax.experimental.pallas import tpu as pltpu
from jax.experimental.pallas import tpu_sc as plsc

BATCH = 16384
EMB = 128

_info = plsc.get_sparse_core_info()
_NC, _NS = _info.num_cores, _info.num_subcores
_B0 = 480
_B1 = (BATCH - _NS * _B0) // _NS

_mesh = plsc.VectorSubcoreMesh(core_axis_name="c", subcore_axis_name="s")


@functools.partial(
    pl.kernel,
    mesh=_mesh,
    out_type=jax.ShapeDtypeStruct((BATCH, EMB), jnp.float32),
    scratch_types=[
        pltpu.VMEM((max(_B0, _B1),), jnp.int32),
        pltpu.VMEM((max(_B0, _B1), EMB), jnp.float32),
        pltpu.SemaphoreType.DMA,
    ],
)
def _gather_kernel(idx_hbm, table_hbm, out_hbm, idx_v, rows_v, sem):
    cid = lax.axis_index("c")
    sid = lax.axis_index("s")

    @pl.when(cid == 0)
    def _():
        base = sid * _B0
        pltpu.sync_copy(idx_hbm.at[pl.ds(base, _B0)], idx_v.at[pl.ds(0, _B0)])
        pltpu.async_copy(
            table_hbm.at[idx_v.at[pl.ds(0, _B0)]],
            rows_v.at[pl.ds(0, _B0)],
            sem,
        ).wait()
        pltpu.sync_copy(rows_v.at[pl.ds(0, _B0)], out_hbm.at[pl.ds(base, _B0)])

    @pl.when(cid == 1)
    def _():
        base = _NS * _B0 + sid * _B1
        pltpu.sync_copy(idx_hbm.at[pl.ds(base, _B1)], idx_v.at[pl.ds(0, _B1)])
        pltpu.async_copy(
            table_hbm.at[idx_v.at[pl.ds(0, _B1)]],
            rows_v.at[pl.ds(0, _B1)],
            sem,
        ).wait()
        pltpu.sync_copy(rows_v.at[pl.ds(0, _B1)], out_hbm.at[pl.ds(base, _B1)])


def kernel(t, pos_embeddings):
    return _gather_kernel(t, pos_embeddings)
